# baseline (speedup 1.0000x reference)
import functools

import jax
import jax.numpy as jnp
from jax import lax
from jax.experimental import pallas as pl
from jax.experimental.pallas import tpu as pltpu
from jax.experimental.pallas import tpu_sc as plsc

E = 320000
N = 10000
DF = 128
DE = 16
MAXN = 10000
NW = 32
EPT = E // NW
BLK = 2048
NBLK = (E + BLK - 1) // BLK
EPAD = NBLK * BLK
CH0 = 1000
GB0 = 40
CS = 80
GBE = 80
SCRPAD = 8192


def _wid():
    return lax.axis_index("s") * 2 + lax.axis_index("c")


def _mesh():
    return plsc.VectorSubcoreMesh(
        core_axis_name="c", subcore_axis_name="s", num_cores=2)


def _gather_body(nodes_ref, idx_hbm, out_hbm, idx_v, rows_v, sem):
    base = _wid() * EPT

    def chunk(k, _):
        off = base + k * CH0
        pltpu.sync_copy(idx_hbm.at[pl.ds(off, CH0)], idx_v)
        descs = [
            pltpu.async_copy(
                nodes_ref.at[idx_v.at[pl.ds(b * GB0, GB0)]],
                rows_v.at[pl.ds(b * GB0, GB0), :],
                sem,
            )
            for b in range(CH0 // GB0)
        ]
        for d in descs:
            d.wait()
        pltpu.sync_copy(rows_v, out_hbm.at[pl.ds(off, CH0), :])
        return 0

    lax.fori_loop(0, EPT // CH0, chunk, 0)


@functools.partial(
    pl.kernel,
    out_type=[
        jax.ShapeDtypeStruct((EPAD, DF), jnp.float32),
        jax.ShapeDtypeStruct((EPAD, DF), jnp.float32),
    ],
    mesh=_mesh(),
    compiler_params=pltpu.CompilerParams(needs_layout_passes=False),
    scratch_types=[
        pltpu.VMEM((CH0,), jnp.int32),
        pltpu.VMEM((CH0, DF), jnp.float32),
        pltpu.SemaphoreType.DMA,
    ],
)
def _k0(nodes_f, rec_hbm, snd_hbm, frec_hbm, fsnd_hbm, idx_v, rows_v, sem):
    _gather_body(nodes_f, rec_hbm, frec_hbm, idx_v, rows_v, sem)
    _gather_body(nodes_f, snd_hbm, fsnd_hbm, idx_v, rows_v, sem)


def _k1_body(frec_ref, fsnd_ref, w_ref, b_ref, noise_ref, ae_ref, out_ref):
    f = jnp.concatenate(
        [frec_ref[...].astype(jnp.float32), fsnd_ref[...].astype(jnp.float32)],
        axis=1,
    )
    logits = jnp.dot(f, w_ref[...], preferred_element_type=jnp.float32) + b_ref[0, 0]
    probs = jax.nn.sigmoid(logits) * ae_ref[...]
    degens = (noise_ref[...] < probs).astype(jnp.float32)
    out_ref[...] = ae_ref[...] * (1.0 - degens)


def _k1(frec_bf, fsnd_bf, w, b2d, noise2d, ae2d):
    return pl.pallas_call(
        _k1_body,
        grid=(NBLK,),
        in_specs=[
            pl.BlockSpec((BLK, DF), lambda i: (i, 0)),
            pl.BlockSpec((BLK, DF), lambda i: (i, 0)),
            pl.BlockSpec((2 * DF, 1), lambda i: (0, 0)),
            pl.BlockSpec((8, 128), lambda i: (0, 0)),
            pl.BlockSpec((BLK, 1), lambda i: (i, 0)),
            pl.BlockSpec((BLK, 1), lambda i: (i, 0)),
        ],
        out_specs=pl.BlockSpec((BLK, 1), lambda i: (i, 0)),
        out_shape=jax.ShapeDtypeStruct((EPAD, 1), jnp.float32),
    )(frec_bf, fsnd_bf, w, b2d, noise2d, ae2d)


@functools.partial(
    pl.kernel,
    out_type=[
        jax.ShapeDtypeStruct((E + SCRPAD,), jnp.int32),
        jax.ShapeDtypeStruct((E + SCRPAD,), jnp.int32),
        jax.ShapeDtypeStruct((E + SCRPAD,), jnp.int32),
        jax.ShapeDtypeStruct((E + SCRPAD,), jnp.int32),
        jax.ShapeDtypeStruct((NW, 16), jnp.int32),
    ],
    mesh=_mesh(),
    compiler_params=pltpu.CompilerParams(needs_layout_passes=False),
    scratch_types=[
        pltpu.VMEM((EPT,), jnp.float32),
        pltpu.VMEM((EPT,), jnp.int32),
        pltpu.VMEM((EPT,), jnp.int32),
        pltpu.VMEM((EPT + 16,), jnp.int32),
        pltpu.VMEM((EPT + 16,), jnp.int32),
        pltpu.VMEM((EPT + 16,), jnp.int32),
        pltpu.VMEM((EPT + 16,), jnp.int32),
        pltpu.VMEM((16,), jnp.int32),
    ],
)
def _b1(nae_hbm, rec_hbm, snd_hbm, ids_c, rec_c, snd_c, nae_c, counts,
        nae_v, rec_v, snd_v, idsc_v, recc_v, sndc_v, naec_v, cnt_row):
    wid = _wid()
    base = wid * EPT
    pltpu.sync_copy(nae_hbm.at[pl.ds(base, EPT)], nae_v)
    pltpu.sync_copy(rec_hbm.at[pl.ds(base, EPT)], rec_v)
    pltpu.sync_copy(snd_hbm.at[pl.ds(base, EPT)], snd_v)

    lane = lax.iota(jnp.int32, 16)

    def group(g, ptr_v):
        off = g * 16
        v = nae_v[pl.ds(off, 16)]
        m = v > 0.0
        mi = m.astype(jnp.int32)
        inc = plsc.cumsum(mi)
        excl = inc - mi
        dst = ptr_v + excl
        plsc.store_scatter(idsc_v, [dst], base + off + lane, mask=m)
        plsc.store_scatter(recc_v, [dst], rec_v[pl.ds(off, 16)], mask=m)
        plsc.store_scatter(sndc_v, [dst], snd_v[pl.ds(off, 16)], mask=m)
        plsc.store_scatter(naec_v, [dst], plsc.bitcast(v, jnp.int32), mask=m)
        return ptr_v + plsc.all_reduce_population_count(m)

    cnt_v = lax.fori_loop(0, EPT // 16, group, jnp.zeros((16,), jnp.int32))

    pltpu.sync_copy(idsc_v.at[pl.ds(0, EPT)], ids_c.at[pl.ds(base, EPT)])
    pltpu.sync_copy(recc_v.at[pl.ds(0, EPT)], rec_c.at[pl.ds(base, EPT)])
    pltpu.sync_copy(sndc_v.at[pl.ds(0, EPT)], snd_c.at[pl.ds(base, EPT)])
    pltpu.sync_copy(naec_v.at[pl.ds(0, EPT)], nae_c.at[pl.ds(base, EPT)])
    cnt_row[pl.ds(0, 16)] = jnp.where(lane == 0, cnt_v, 0)
    pltpu.sync_copy(cnt_row, counts.at[wid])


STG = 4096
WIN = 4080


@functools.partial(
    pl.kernel,
    out_type=[
        jax.ShapeDtypeStruct((E,), jnp.int32),
        jax.ShapeDtypeStruct((E,), jnp.int32),
        jax.ShapeDtypeStruct((E,), jnp.int32),
        jax.ShapeDtypeStruct((E, DE), jnp.float32),
    ],
    mesh=_mesh(),
    compiler_params=pltpu.CompilerParams(needs_layout_passes=False),
    scratch_types=[
        pltpu.VMEM((512,), jnp.int32),
        pltpu.VMEM((EPT + 16,), jnp.int32),
        pltpu.VMEM((EPT + 16,), jnp.int32),
        pltpu.VMEM((EPT + 16,), jnp.int32),
        pltpu.VMEM((EPT + 16,), jnp.int32),
        pltpu.VMEM((STG,), jnp.int32),
        pltpu.VMEM((CS,), jnp.int32),
        pltpu.VMEM((CS, 8 * DE), jnp.float32),
        pltpu.VMEM((CS, DE), jnp.float32),
        pltpu.SemaphoreType.DMA,
    ],
)
def _b2(counts_flat, ids_c, rec_c, snd_c, nae_c, edges_sr,
        naedges_o, nsend_o, nrec_o, edges_o,
        cvm, seg_nae, seg_rec, seg_snd, seg_ids, stg, supb, srows_v, out_v, sem):
    wid = _wid()
    lo = wid * EPT
    lane = lax.iota(jnp.int32, 16)

    pltpu.sync_copy(counts_flat, cvm)

    def addc(j, a):
        return a + cvm[pl.ds(j * 16, 16)][0]

    ktot = lax.fori_loop(0, NW, addc, jnp.int32(0))
    sur_hi = jnp.minimum(lo + EPT, ktot)

    zero_bits = jnp.int32(0)
    sentinel = jnp.int32(MAXN - 1)

    def init(g, _):
        off = g * 16
        seg_nae[pl.ds(off, 16)] = jnp.full((16,), zero_bits, jnp.int32)
        seg_rec[pl.ds(off, 16)] = jnp.full((16,), sentinel, jnp.int32)
        seg_snd[pl.ds(off, 16)] = jnp.full((16,), sentinel, jnp.int32)
        seg_ids[pl.ds(off, 16)] = jnp.full((16,), jnp.int32(E), jnp.int32)
        return 0

    lax.fori_loop(0, (EPT + 16) // 16, init, 0)

    def fill_from(src_hbm, seg_ref, p, s, length):
        def window(w, _):
            p2 = p + w * WIN
            l2 = jnp.minimum(length - w * WIN, WIN)
            p2_al = (p2 // 8) * 8
            shift = p2 - p2_al

            def dma_chunk(k, _):
                pltpu.sync_copy(src_hbm.at[pl.ds(p2_al + k * 1024, 1024)],
                                stg.at[pl.ds(k * 1024, 1024)])
                return 0

            lax.fori_loop(0, STG // 1024, dma_chunk, 0)
            dbase = (s - lo) + w * WIN

            def fgroup(g, _):
                o = g * 16
                sv = plsc.load_gather(stg, [shift + o + lane])
                di = dbase + o + lane
                msk = (o + lane) < l2
                plsc.store_scatter(seg_ref, [di], sv, mask=msk)
                return 0

            lax.fori_loop(0, (l2 + 15) // 16, fgroup, 0)
            return 0

        lax.fori_loop(0, (length + WIN - 1) // WIN, window, 0)

    def run_j(j, cum):
        cj = cvm[pl.ds(j * 16, 16)][0]
        s = jnp.maximum(cum, lo)
        e = jnp.minimum(cum + cj, sur_hi)

        @pl.when(e > s)
        def _():
            p = j * EPT + (s - cum)
            ln = e - s
            fill_from(ids_c, seg_ids, p, s, ln)
            fill_from(rec_c, seg_rec, p, s, ln)
            fill_from(snd_c, seg_snd, p, s, ln)
            fill_from(nae_c, seg_nae, p, s, ln)

        return cum + cj

    lax.fori_loop(0, NW, run_j, jnp.int32(0))

    pltpu.sync_copy(seg_nae.at[pl.ds(0, EPT)], naedges_o.at[pl.ds(lo, EPT)])
    pltpu.sync_copy(seg_snd.at[pl.ds(0, EPT)], nsend_o.at[pl.ds(lo, EPT)])
    pltpu.sync_copy(seg_rec.at[pl.ds(0, EPT)], nrec_o.at[pl.ds(lo, EPT)])

    def echunk(k, _):
        def mk_sup(g, _):
            ids16 = seg_ids[pl.ds(k * CS + g * 16, 16)]
            supb[pl.ds(g * 16, 16)] = lax.shift_right_logical(ids16, 3)
            return 0

        lax.fori_loop(0, CS // 16, mk_sup, 0)
        descs = [
            pltpu.async_copy(
                edges_sr.at[supb.at[pl.ds(b * GBE, GBE)]],
                srows_v.at[pl.ds(b * GBE, GBE), :],
                sem,
            )
            for b in range(CS // GBE)
        ]
        for d in descs:
            d.wait()

        def extract(g, _):
            r_v = g * 16 + lane
            ids16 = seg_ids[pl.ds(k * CS + g * 16, 16)]
            colbase = (ids16 & 7) * DE
            for c in range(DE):
                vals = plsc.load_gather(srows_v, [r_v, colbase + c])
                plsc.store_scatter(out_v, [r_v, jnp.full((16,), c, jnp.int32)], vals)
            return 0

        lax.fori_loop(0, CS // 16, extract, 0)
        pltpu.sync_copy(out_v, edges_o.at[pl.ds(lo + k * CS, CS), :])
        return 0

    lax.fori_loop(0, EPT // CS, echunk, 0)


def kernel(nodes, edges, receivers, senders, active_nodes, active_edges,
           uniform_noise, W_prob, b_prob):
    frec_f, fsnd_f = _k0(nodes, receivers, senders)

    pad = EPAD - E
    noise2d = jnp.pad(uniform_noise, (0, pad)).reshape(EPAD, 1)
    ae2d = jnp.pad(active_edges, (0, pad)).reshape(EPAD, 1)
    b2d = jnp.full((8, 128), b_prob[0], jnp.float32)

    nae_pre = _k1(frec_f, fsnd_f, W_prob, b2d, noise2d, ae2d)[:E, 0]

    ids_c, rec_c, snd_c, nae_c, counts = _b1(nae_pre, receivers, senders)

    edges_sr = jnp.concatenate(
        [edges, jnp.zeros((8, DE), jnp.float32)], axis=0).reshape((E + 8) // 8, 8 * DE)
    counts_flat = counts.reshape(NW * 16)

    nae_bits, nsend, nrec, new_edges = _b2(
        counts_flat, ids_c, rec_c, snd_c, nae_c, edges_sr)
    naedges = lax.bitcast_convert_type(nae_bits, jnp.float32)
    return naedges, nsend, nrec, new_edges


# speedup vs baseline: 1.0006x; 1.0006x over previous
import functools

import jax
import jax.numpy as jnp
from jax import lax
from jax.experimental import pallas as pl
from jax.experimental.pallas import tpu as pltpu
from jax.experimental.pallas import tpu_sc as plsc

E = 320000
N = 10000
DF = 128
DE = 16
MAXN = 10000
NW = 32
EPT = E // NW
BLK = 2048
NBLK = (E + BLK - 1) // BLK
EPAD = NBLK * BLK
CH0 = 1000
GB0 = 40
CS = 80
GBE = 80
SCRPAD = 8192


def _wid():
    return lax.axis_index("s") * 2 + lax.axis_index("c")


def _mesh():
    return plsc.VectorSubcoreMesh(
        core_axis_name="c", subcore_axis_name="s", num_cores=2)


def _gather_body(nodes_ref, idx_hbm, out_hbm, idx_v, rows_v, sem):
    base = _wid() * EPT

    def chunk(k, _):
        off = base + k * CH0
        pltpu.sync_copy(idx_hbm.at[pl.ds(off, CH0)], idx_v)
        descs = [
            pltpu.async_copy(
                nodes_ref.at[idx_v.at[pl.ds(b * GB0, GB0)]],
                rows_v.at[pl.ds(b * GB0, GB0), :],
                sem,
            )
            for b in range(CH0 // GB0)
        ]
        for d in descs:
            d.wait()
        pltpu.sync_copy(rows_v, out_hbm.at[pl.ds(off, CH0), :])
        return 0

    lax.fori_loop(0, EPT // CH0, chunk, 0)


@functools.partial(
    pl.kernel,
    out_type=[
        jax.ShapeDtypeStruct((EPAD, DF), jnp.float32),
        jax.ShapeDtypeStruct((EPAD, DF), jnp.float32),
    ],
    mesh=_mesh(),
    compiler_params=pltpu.CompilerParams(needs_layout_passes=False),
    scratch_types=[
        pltpu.VMEM((CH0,), jnp.int32),
        pltpu.VMEM((CH0, DF), jnp.float32),
        pltpu.SemaphoreType.DMA,
    ],
)
def _k0(nodes_f, rec_hbm, snd_hbm, frec_hbm, fsnd_hbm, idx_v, rows_v, sem):
    _gather_body(nodes_f, rec_hbm, frec_hbm, idx_v, rows_v, sem)
    _gather_body(nodes_f, snd_hbm, fsnd_hbm, idx_v, rows_v, sem)


def _k1_body(frec_ref, fsnd_ref, w_ref, b_ref, noise_ref, ae_ref, out_ref):
    f = jnp.concatenate(
        [frec_ref[...].astype(jnp.float32), fsnd_ref[...].astype(jnp.float32)],
        axis=1,
    )
    logits = jnp.dot(f, w_ref[...], preferred_element_type=jnp.float32) + b_ref[0, 0]
    probs = jax.nn.sigmoid(logits) * ae_ref[...]
    degens = (noise_ref[...] < probs).astype(jnp.float32)
    out_ref[...] = ae_ref[...] * (1.0 - degens)


def _k1(frec_bf, fsnd_bf, w, b2d, noise2d, ae2d):
    return pl.pallas_call(
        _k1_body,
        grid=(NBLK,),
        in_specs=[
            pl.BlockSpec((BLK, DF), lambda i: (i, 0)),
            pl.BlockSpec((BLK, DF), lambda i: (i, 0)),
            pl.BlockSpec((2 * DF, 1), lambda i: (0, 0)),
            pl.BlockSpec((8, 128), lambda i: (0, 0)),
            pl.BlockSpec((BLK, 1), lambda i: (i, 0)),
            pl.BlockSpec((BLK, 1), lambda i: (i, 0)),
        ],
        out_specs=pl.BlockSpec((BLK, 1), lambda i: (i, 0)),
        out_shape=jax.ShapeDtypeStruct((EPAD, 1), jnp.float32),
    )(frec_bf, fsnd_bf, w, b2d, noise2d, ae2d)


@functools.partial(
    pl.kernel,
    out_type=[
        jax.ShapeDtypeStruct((E + SCRPAD,), jnp.int32),
        jax.ShapeDtypeStruct((E + SCRPAD,), jnp.int32),
        jax.ShapeDtypeStruct((E + SCRPAD,), jnp.int32),
        jax.ShapeDtypeStruct((E + SCRPAD,), jnp.int32),
        jax.ShapeDtypeStruct((NW, 16), jnp.int32),
    ],
    mesh=_mesh(),
    compiler_params=pltpu.CompilerParams(needs_layout_passes=False),
    scratch_types=[
        pltpu.VMEM((EPT,), jnp.float32),
        pltpu.VMEM((EPT,), jnp.int32),
        pltpu.VMEM((EPT,), jnp.int32),
        pltpu.VMEM((EPT + 16,), jnp.int32),
        pltpu.VMEM((EPT + 16,), jnp.int32),
        pltpu.VMEM((EPT + 16,), jnp.int32),
        pltpu.VMEM((EPT + 16,), jnp.int32),
        pltpu.VMEM((16,), jnp.int32),
    ],
)
def _b1(nae_hbm, rec_hbm, snd_hbm, ids_c, rec_c, snd_c, nae_c, counts,
        nae_v, rec_v, snd_v, idsc_v, recc_v, sndc_v, naec_v, cnt_row):
    wid = _wid()
    base = wid * EPT
    pltpu.sync_copy(nae_hbm.at[pl.ds(base, EPT)], nae_v)
    pltpu.sync_copy(rec_hbm.at[pl.ds(base, EPT)], rec_v)
    pltpu.sync_copy(snd_hbm.at[pl.ds(base, EPT)], snd_v)

    lane = lax.iota(jnp.int32, 16)

    def group(g, ptr_v):
        off = g * 16
        v = nae_v[pl.ds(off, 16)]
        m = v > 0.0
        mi = m.astype(jnp.int32)
        inc = plsc.cumsum(mi)
        excl = inc - mi
        dst = ptr_v + excl
        plsc.store_scatter(idsc_v, [dst], base + off + lane, mask=m)
        plsc.store_scatter(recc_v, [dst], rec_v[pl.ds(off, 16)], mask=m)
        plsc.store_scatter(sndc_v, [dst], snd_v[pl.ds(off, 16)], mask=m)
        plsc.store_scatter(naec_v, [dst], plsc.bitcast(v, jnp.int32), mask=m)
        return ptr_v + plsc.all_reduce_population_count(m)

    cnt_v = lax.fori_loop(0, EPT // 16, group, jnp.zeros((16,), jnp.int32))

    pltpu.sync_copy(idsc_v.at[pl.ds(0, EPT)], ids_c.at[pl.ds(base, EPT)])
    pltpu.sync_copy(recc_v.at[pl.ds(0, EPT)], rec_c.at[pl.ds(base, EPT)])
    pltpu.sync_copy(sndc_v.at[pl.ds(0, EPT)], snd_c.at[pl.ds(base, EPT)])
    pltpu.sync_copy(naec_v.at[pl.ds(0, EPT)], nae_c.at[pl.ds(base, EPT)])
    cnt_row[pl.ds(0, 16)] = jnp.where(lane == 0, cnt_v, 0)
    pltpu.sync_copy(cnt_row, counts.at[wid])


STG = 4096
WIN = 4080


@functools.partial(
    pl.kernel,
    out_type=[
        jax.ShapeDtypeStruct((E,), jnp.int32),
        jax.ShapeDtypeStruct((E,), jnp.int32),
        jax.ShapeDtypeStruct((E,), jnp.int32),
        jax.ShapeDtypeStruct((E, DE), jnp.float32),
    ],
    mesh=_mesh(),
    compiler_params=pltpu.CompilerParams(needs_layout_passes=False),
    scratch_types=[
        pltpu.VMEM((512,), jnp.int32),
        pltpu.VMEM((EPT + 16,), jnp.int32),
        pltpu.VMEM((EPT + 16,), jnp.int32),
        pltpu.VMEM((EPT + 16,), jnp.int32),
        pltpu.VMEM((EPT + 16,), jnp.int32),
        pltpu.VMEM((STG,), jnp.int32),
        pltpu.VMEM((CS,), jnp.int32),
        pltpu.VMEM((CS,), jnp.int32),
        pltpu.VMEM((CS, 8 * DE), jnp.float32),
        pltpu.VMEM((CS, 8 * DE), jnp.float32),
        pltpu.VMEM((CS, DE), jnp.float32),
        pltpu.VMEM((CS, DE), jnp.float32),
        pltpu.SemaphoreType.DMA,
        pltpu.SemaphoreType.DMA,
    ],
)
def _b2(counts_flat, ids_c, rec_c, snd_c, nae_c, edges_sr,
        naedges_o, nsend_o, nrec_o, edges_o,
        cvm, seg_nae, seg_rec, seg_snd, seg_ids, stg,
        supb0, supb1, srows0, srows1, outv0, outv1, sem, osem):
    wid = _wid()
    lo = wid * EPT
    lane = lax.iota(jnp.int32, 16)

    pltpu.sync_copy(counts_flat, cvm)

    def addc(j, a):
        return a + cvm[pl.ds(j * 16, 16)][0]

    ktot = lax.fori_loop(0, NW, addc, jnp.int32(0))
    sur_hi = jnp.minimum(lo + EPT, ktot)

    zero_bits = jnp.int32(0)
    sentinel = jnp.int32(MAXN - 1)

    def init(g, _):
        off = g * 16
        seg_nae[pl.ds(off, 16)] = jnp.full((16,), zero_bits, jnp.int32)
        seg_rec[pl.ds(off, 16)] = jnp.full((16,), sentinel, jnp.int32)
        seg_snd[pl.ds(off, 16)] = jnp.full((16,), sentinel, jnp.int32)
        seg_ids[pl.ds(off, 16)] = jnp.full((16,), jnp.int32(E), jnp.int32)
        return 0

    lax.fori_loop(0, (EPT + 16) // 16, init, 0)

    def fill_from(src_hbm, seg_ref, p, s, length):
        def window(w, _):
            p2 = p + w * WIN
            l2 = jnp.minimum(length - w * WIN, WIN)
            p2_al = (p2 // 8) * 8
            shift = p2 - p2_al

            def dma_chunk(k, _):
                pltpu.sync_copy(src_hbm.at[pl.ds(p2_al + k * 1024, 1024)],
                                stg.at[pl.ds(k * 1024, 1024)])
                return 0

            lax.fori_loop(0, STG // 1024, dma_chunk, 0)
            dbase = (s - lo) + w * WIN

            def fgroup(g, _):
                o = g * 16
                sv = plsc.load_gather(stg, [shift + o + lane])
                di = dbase + o + lane
                msk = (o + lane) < l2
                plsc.store_scatter(seg_ref, [di], sv, mask=msk)
                return 0

            lax.fori_loop(0, (l2 + 15) // 16, fgroup, 0)
            return 0

        lax.fori_loop(0, (length + WIN - 1) // WIN, window, 0)

    def run_j(j, cum):
        cj = cvm[pl.ds(j * 16, 16)][0]
        s = jnp.maximum(cum, lo)
        e = jnp.minimum(cum + cj, sur_hi)

        @pl.when(e > s)
        def _():
            p = j * EPT + (s - cum)
            ln = e - s
            fill_from(ids_c, seg_ids, p, s, ln)
            fill_from(rec_c, seg_rec, p, s, ln)
            fill_from(snd_c, seg_snd, p, s, ln)
            fill_from(nae_c, seg_nae, p, s, ln)

        return cum + cj

    lax.fori_loop(0, NW, run_j, jnp.int32(0))

    pltpu.sync_copy(seg_nae.at[pl.ds(0, EPT)], naedges_o.at[pl.ds(lo, EPT)])
    pltpu.sync_copy(seg_snd.at[pl.ds(0, EPT)], nsend_o.at[pl.ds(lo, EPT)])
    pltpu.sync_copy(seg_rec.at[pl.ds(0, EPT)], nrec_o.at[pl.ds(lo, EPT)])

    NCHK = EPT // CS

    def issue(k, supb, srows):
        def mk_sup(g, _):
            ids16 = seg_ids[pl.ds(k * CS + g * 16, 16)]
            supb[pl.ds(g * 16, 16)] = lax.shift_right_logical(ids16, 3)
            return 0

        lax.fori_loop(0, CS // 16, mk_sup, 0)
        pltpu.async_copy(edges_sr.at[supb], srows, sem)

    def process(k, supb, srows, out_v, first):
        pltpu.make_async_copy(edges_sr.at[supb], srows, sem).wait()

        def extract(g, _):
            r_v = g * 16 + lane
            ids16 = seg_ids[pl.ds(k * CS + g * 16, 16)]
            colbase = (ids16 & 7) * DE
            for c in range(DE):
                vals = plsc.load_gather(srows, [r_v, colbase + c])
                plsc.store_scatter(out_v, [r_v, jnp.full((16,), c, jnp.int32)], vals)
            return 0

        lax.fori_loop(0, CS // 16, extract, 0)

        @pl.when(jnp.logical_not(first))
        def _():
            pltpu.make_async_copy(
                out_v, edges_o.at[pl.ds(lo, CS), :], osem).wait()

        pltpu.async_copy(out_v, edges_o.at[pl.ds(lo + k * CS, CS), :], osem)

    issue(0, supb0, srows0)

    def echunk(k, _):
        even = (k % 2) == 0

        @pl.when(even)
        def _():
            @pl.when(k + 1 < NCHK)
            def _():
                issue(k + 1, supb1, srows1)

            process(k, supb0, srows0, outv0, k < 2)

        @pl.when(jnp.logical_not(even))
        def _():
            @pl.when(k + 1 < NCHK)
            def _():
                issue(k + 1, supb0, srows0)

            process(k, supb1, srows1, outv1, k < 2)

        return 0

    lax.fori_loop(0, NCHK, echunk, 0)
    pltpu.make_async_copy(outv0, edges_o.at[pl.ds(lo, CS), :], osem).wait()
    pltpu.make_async_copy(outv1, edges_o.at[pl.ds(lo, CS), :], osem).wait()


def kernel(nodes, edges, receivers, senders, active_nodes, active_edges,
           uniform_noise, W_prob, b_prob):
    frec_f, fsnd_f = _k0(nodes, receivers, senders)

    pad = EPAD - E
    noise2d = jnp.pad(uniform_noise, (0, pad)).reshape(EPAD, 1)
    ae2d = jnp.pad(active_edges, (0, pad)).reshape(EPAD, 1)
    b2d = jnp.full((8, 128), b_prob[0], jnp.float32)

    nae_pre = _k1(frec_f, fsnd_f, W_prob, b2d, noise2d, ae2d)[:E, 0]

    ids_c, rec_c, snd_c, nae_c, counts = _b1(nae_pre, receivers, senders)

    edges_sr = jnp.concatenate(
        [edges, jnp.zeros((8, DE), jnp.float32)], axis=0).reshape((E + 8) // 8, 8 * DE)
    counts_flat = counts.reshape(NW * 16)

    nae_bits, nsend, nrec, new_edges = _b2(
        counts_flat, ids_c, rec_c, snd_c, nae_c, edges_sr)
    naedges = lax.bitcast_convert_type(nae_bits, jnp.float32)
    return naedges, nsend, nrec, new_edges
